# baseline (speedup 1.0000x reference)
import functools

import jax
import jax.numpy as jnp
from jax import lax
from jax.experimental import pallas as pl
from jax.experimental.pallas import tpu as pltpu
from jax.experimental.pallas import tpu_sc as plsc

N = 10000
HID = 32
DEGW = 16

NC = 2
NS = 16
NW = NC * NS
K = 128
NBUF = 8
HALF = NBUF // 2

N_PAD = 10240
RPT = N_PAD // NS
FN = N_PAD * HID // 128
PIECES = RPT // K

_mesh = plsc.VectorSubcoreMesh(core_axis_name="c", subcore_axis_name="s")


def _zero_rows(ref, nrows, ncols):
    zero = jnp.zeros((16,), jnp.float32)

    def body(i, _):
        for c in range(ncols // 16):
            ref[i, pl.ds(c * 16, 16)] = zero
        return 0

    lax.fori_loop(0, nrows, body, 0)


PREP_BK = 8192


def _tc_prep(edge_index, nchunk):
    e = edge_index.shape[1]
    e_pad = NW * nchunk * K
    lanes = e_pad // 8
    ed3 = edge_index.reshape(2, 1, e)

    def body(row_ref, col_ref, rout_ref, cout_ref):
        r = pl.program_id(0)
        l = pl.program_id(1)
        base = (r * (lanes // PREP_BK) + l) * PREP_BK
        pos = base + lax.broadcasted_iota(jnp.int32, (1, 1, PREP_BK), 2)
        keep = pos < e
        rout_ref[...] = jnp.where(keep, row_ref[...], N)
        cout_ref[...] = jnp.where(keep, col_ref[...], N)

    blk = lambda r, l: (0, 0, r * (lanes // PREP_BK) + l)
    out_blk = lambda r, l: (r, 0, l)
    shp = jax.ShapeDtypeStruct((8, 1, lanes), jnp.int32)
    return pl.pallas_call(
        body,
        grid=(8, lanes // PREP_BK),
        in_specs=[
            pl.BlockSpec((1, 1, PREP_BK), blk),
            pl.BlockSpec((1, 1, PREP_BK), blk),
        ],
        out_specs=[
            pl.BlockSpec((1, 1, PREP_BK), out_blk),
            pl.BlockSpec((1, 1, PREP_BK), out_blk),
        ],
        out_shape=[shp, shp],
    )(ed3[0:1], ed3[1:2])


def _load_col(cp_hbm, col_v, wid, nchunk, sem):
    r = wid // 4
    l = (wid % 4) * (nchunk * K)

    def issue(j, _):
        pltpu.async_copy(
            cp_hbm.at[r, 0, pl.ds(l + j * K, K)], col_v.at[j], sem)
        return 0

    lax.fori_loop(0, nchunk, issue, 0)

    def drain(j, _):
        pltpu.make_async_copy(
            cp_hbm.at[r, 0, pl.ds(l, K)], col_v.at[0], sem).wait()
        return 0

    lax.fori_loop(0, nchunk, drain, 0)


def _load_row(rp_hbm, row_v, wid, nchunk):
    r = wid // 4
    l = (wid % 4) * (nchunk * K)
    pltpu.sync_copy(rp_hbm.at[r, 0, pl.ds(l, nchunk * K)], row_v)


def _make_deg_kernel(nchunk):
    @functools.partial(
        pl.kernel,
        out_type=jax.ShapeDtypeStruct((NC, N_PAD, DEGW), jnp.float32),
        mesh=_mesh,
        scratch_types=[
            pltpu.VMEM((nchunk, K), jnp.int32),
            pltpu.VMEM((K, DEGW), jnp.float32),
            pltpu.VMEM((RPT, DEGW), jnp.float32),
            pltpu.VMEM_SHARED((N_PAD, DEGW), jnp.float32),
            pltpu.SemaphoreType.DMA,
        ],
        compiler_params=pltpu.CompilerParams(use_tc_tiling_on_sc=False),
    )
    def deg_kernel(cp_hbm, degp_hbm, col_v, ones_v, zrow_v, deg_sh, dsem):
        cid = lax.axis_index("c")
        sid = lax.axis_index("s")
        wid = sid * NC + cid
        _load_col(cp_hbm, col_v, wid, nchunk, dsem)

        one = jnp.full((16,), 1.0, jnp.float32)

        def fill_ones(i, _):
            ones_v[i, :] = one
            return 0

        lax.fori_loop(0, K, fill_ones, 0)
        _zero_rows(zrow_v, RPT, DEGW)

        pltpu.sync_copy(zrow_v, deg_sh.at[pl.ds(sid * RPT, RPT)])
        plsc.subcore_barrier()

        for j in range(8):
            pltpu.async_copy(ones_v, deg_sh.at[col_v.at[j]], dsem, add=True)

        def chunk(h, _):
            for j in range(8):
                s = 8 * h + j
                pltpu.make_async_copy(
                    ones_v, deg_sh.at[col_v.at[s]], dsem).wait()
                pltpu.async_copy(
                    ones_v, deg_sh.at[col_v.at[s + 8]], dsem, add=True)
            return 0

        lax.fori_loop(0, nchunk // 8 - 1, chunk, 0)
        for j in range(8):
            pltpu.make_async_copy(
                ones_v, deg_sh.at[col_v.at[j]], dsem).wait()
        plsc.subcore_barrier()

        pltpu.sync_copy(
            deg_sh.at[pl.ds(sid * RPT, RPT)],
            degp_hbm.at[cid, pl.ds(sid * RPT, RPT)],
        )

    return deg_kernel


def _ring(feat_sh, agg_sh, row_v, col_v, bufs, zrow_v, gsem, ssem, nchunk):
    def ridx(s):
        return row_v.at[pl.ds(s * K, K)]

    zsrc = zrow_v.at[pl.ds(0, K)]
    for j in range(HALF):
        pltpu.async_copy(
            zsrc, agg_sh.at[col_v.at[0]], ssem[HALF + j], add=True)
    for j in range(HALF):
        pltpu.async_copy(feat_sh.at[ridx(j)], bufs[j], gsem[j])

    def step(h, _):
        for j in range(NBUF):
            s = NBUF * h + j
            pltpu.make_async_copy(
                feat_sh.at[ridx(s)], bufs[j], gsem[j]).wait()
            pltpu.async_copy(
                bufs[j], agg_sh.at[col_v.at[s]], ssem[j], add=True)
            bb = (j + HALF) % NBUF
            pltpu.make_async_copy(
                bufs[bb], agg_sh.at[col_v.at[s]], ssem[bb]).wait()
            s2 = (s + HALF) % nchunk
            pltpu.async_copy(feat_sh.at[ridx(s2)], bufs[bb], gsem[bb])
        return 0

    lax.fori_loop(0, nchunk // NBUF, step, 0)

    for j in range(HALF):
        pltpu.make_async_copy(
            bufs[HALF + j], agg_sh.at[col_v.at[0]], ssem[HALF + j]).wait()
        pltpu.make_async_copy(
            feat_sh.at[ridx(j)], bufs[j], gsem[j]).wait()


def _make_agg1_kernel(nchunk):
    @functools.partial(
        pl.kernel,
        out_type=jax.ShapeDtypeStruct((NC, N_PAD, HID), jnp.float32),
        mesh=_mesh,
        scratch_types=[
            pltpu.VMEM((nchunk * K,), jnp.int32),
            pltpu.VMEM((nchunk, K), jnp.int32),
            [pltpu.VMEM((K, HID), jnp.float32) for _ in range(NBUF)],
            pltpu.VMEM((RPT, HID), jnp.float32),
            pltpu.VMEM_SHARED((N_PAD, HID), jnp.float32),
            pltpu.VMEM_SHARED((N_PAD, HID), jnp.float32),
            [pltpu.SemaphoreType.DMA for _ in range(NBUF)],
            [pltpu.SemaphoreType.DMA for _ in range(NBUF)],
        ],
        compiler_params=pltpu.CompilerParams(use_tc_tiling_on_sc=False),
    )
    def agg_kernel(feat_hbm, rp_hbm, cp_hbm, aggp_hbm,
                   row_v, col_v, bufs, zrow_v, agg_sh, feat_sh, gsem, ssem):
        cid = lax.axis_index("c")
        sid = lax.axis_index("s")
        wid = sid * NC + cid

        _load_row(rp_hbm, row_v, wid, nchunk)
        _load_col(cp_hbm, col_v, wid, nchunk, gsem[0])

        pltpu.sync_copy(
            feat_hbm.at[pl.ds(sid * RPT, RPT)],
            feat_sh.at[pl.ds(sid * RPT, RPT)],
        )

        _zero_rows(zrow_v, RPT, HID)
        pltpu.sync_copy(zrow_v, agg_sh.at[pl.ds(sid * RPT, RPT)])
        plsc.subcore_barrier()

        _ring(feat_sh, agg_sh, row_v, col_v, bufs, zrow_v, gsem, ssem, nchunk)
        plsc.subcore_barrier()

        pltpu.sync_copy(
            agg_sh.at[pl.ds(sid * RPT, RPT)],
            aggp_hbm.at[cid, pl.ds(sid * RPT, RPT)],
        )

    return agg_kernel


def _make_agg2_kernel(nchunk):
    @functools.partial(
        pl.kernel,
        out_type=jax.ShapeDtypeStruct((NC, N_PAD, HID), jnp.float32),
        mesh=_mesh,
        scratch_types=[
            pltpu.VMEM((nchunk * K,), jnp.int32),
            pltpu.VMEM((nchunk, K), jnp.int32),
            [pltpu.VMEM((K, HID), jnp.float32) for _ in range(NBUF)],
            pltpu.VMEM((RPT, HID), jnp.float32),
            pltpu.VMEM((HID,), jnp.float32),
            pltpu.VMEM_SHARED((N_PAD, HID), jnp.float32),
            pltpu.VMEM_SHARED((N_PAD, HID), jnp.float32),
            [pltpu.SemaphoreType.DMA for _ in range(NBUF)],
            [pltpu.SemaphoreType.DMA for _ in range(NBUF)],
        ],
        compiler_params=pltpu.CompilerParams(use_tc_tiling_on_sc=False),
    )
    def agg_kernel(aggp_in_hbm, dinvw_hbm, b1_hbm, rp_hbm, cp_hbm,
                   aggp_hbm, row_v, col_v, bufs, zrow_v, b1_v,
                   agg_sh, feat_sh, gsem, ssem):
        cid = lax.axis_index("c")
        sid = lax.axis_index("s")
        wid = sid * NC + cid

        _load_row(rp_hbm, row_v, wid, nchunk)
        _load_col(cp_hbm, col_v, wid, nchunk, gsem[0])
        pltpu.sync_copy(b1_hbm, b1_v)

        def piece(p, _):
            base = sid * RPT + p * K
            pltpu.sync_copy(aggp_in_hbm.at[0, pl.ds(base, K)], bufs[0])
            pltpu.sync_copy(aggp_in_hbm.at[1, pl.ds(base, K)], bufs[1])
            pltpu.sync_copy(dinvw_hbm.at[pl.ds(base, K)], bufs[2])

            def row(i, _):
                for c in range(HID // 16):
                    sl = pl.ds(c * 16, 16)
                    d = bufs[2][i, sl]
                    a = bufs[0][i, sl] + bufs[1][i, sl]
                    h = jnp.maximum(a * d + b1_v[sl], 0.0)
                    bufs[3][i, sl] = h * d
                return 0

            lax.fori_loop(0, K, row, 0)
            pltpu.sync_copy(bufs[3], feat_sh.at[pl.ds(base, K)])
            return 0

        lax.fori_loop(0, PIECES, piece, 0)

        _zero_rows(zrow_v, RPT, HID)
        pltpu.sync_copy(zrow_v, agg_sh.at[pl.ds(sid * RPT, RPT)])
        plsc.subcore_barrier()

        _ring(feat_sh, agg_sh, row_v, col_v, bufs, zrow_v, gsem, ssem, nchunk)
        plsc.subcore_barrier()

        pltpu.sync_copy(
            agg_sh.at[pl.ds(sid * RPT, RPT)],
            aggp_hbm.at[cid, pl.ds(sid * RPT, RPT)],
        )

    return agg_kernel


def _tc_xw_scale(x, w1, degp):
    def body(x_ref, w_ref, degp_ref, t1_ref, dinv_ref):
        deg = degp_ref[0, :, 0:1] + degp_ref[1, :, 0:1]
        rows = lax.broadcasted_iota(jnp.int32, (N_PAD, 1), 0)
        dinv = jnp.where(
            (deg > 0) & (rows < N),
            lax.rsqrt(jnp.maximum(deg, 1e-12)), 0.0)
        xw = jnp.dot(x_ref[...], w_ref[...],
                     preferred_element_type=jnp.float32)
        xw_pad = jnp.concatenate(
            [xw, jnp.zeros((N_PAD - N, HID), jnp.float32)], axis=0)
        t1_ref[...] = xw_pad * dinv
        dinv_ref[...] = jnp.broadcast_to(dinv, (N_PAD, HID))

    return pl.pallas_call(
        body,
        out_shape=(
            jax.ShapeDtypeStruct((N_PAD, HID), jnp.float32),
            jax.ShapeDtypeStruct((N_PAD, HID), jnp.float32),
        ),
    )(x, w1, degp)


def _tc_final(aggp_f, dinvw_f, w2, b2):
    def body(aggp_ref, dinv_ref, w2_ref, b2_ref, out_ref):
        agg = aggp_ref[0] + aggp_ref[1]
        dv = dinv_ref[...]
        y = jnp.dot((agg * dv)[:N, :], w2_ref[...],
                    preferred_element_type=jnp.float32) + b2_ref[...]
        m = jnp.max(y, axis=1, keepdims=True)
        s = y - m
        lse = jnp.log(jnp.sum(jnp.exp(s), axis=1, keepdims=True))
        out_ref[...] = s - lse

    return pl.pallas_call(
        body,
        out_shape=jax.ShapeDtypeStruct((N, 128), jnp.float32),
    )(aggp_f, dinvw_f, w2, b2)


def kernel(x, edge_index, W1, b1, W2, b2):
    e = edge_index.shape[1]
    nchunk = -(-e // (NW * K))
    nchunk = -(-nchunk // NBUF) * NBUF

    row_p, col_p = _tc_prep(edge_index.astype(jnp.int32), nchunk)

    degp = _make_deg_kernel(nchunk)(col_p)
    t1s, dinvw = _tc_xw_scale(x, W1, degp)
    agg1p = _make_agg1_kernel(nchunk)(t1s, row_p, col_p)
    agg2p = _make_agg2_kernel(nchunk)(agg1p, dinvw, b1, row_p, col_p)
    return _tc_final(agg2p, dinvw, W2, b2)


# speedup vs baseline: 1.1336x; 1.1336x over previous
import functools

import jax
import jax.numpy as jnp
from jax import lax
from jax.experimental import pallas as pl
from jax.experimental.pallas import tpu as pltpu
from jax.experimental.pallas import tpu_sc as plsc

N = 10000
HID = 32
DEGW = 16

NC = 2
NS = 16
NW = NC * NS
K = 128
NBUF = 8
HALF = NBUF // 2

N_PAD = 10240
RPT = N_PAD // NS
FN = N_PAD * HID // 128
PIECES = RPT // K

_mesh = plsc.VectorSubcoreMesh(core_axis_name="c", subcore_axis_name="s")


def _zero_rows(ref, nrows, ncols):
    zero = jnp.zeros((16,), jnp.float32)

    def body(i, _):
        for c in range(ncols // 16):
            ref[i, pl.ds(c * 16, 16)] = zero
        return 0

    lax.fori_loop(0, nrows, body, 0)


def _nreal(e, wid, nchunk):
    return jnp.clip(e // K - wid * nchunk, 0, nchunk)


def _fire_col(col_hbm, col_v, wid, nchunk, nreal, sem):
    base0 = wid * nchunk * K

    def issue(j, _):
        pltpu.async_copy(
            col_hbm.at[pl.ds(base0 + j * K, K)], col_v.at[j], sem)
        return 0

    lax.fori_loop(0, nreal, issue, 0)

    padv = jnp.full((16,), N, jnp.int32)

    def fill(j, _):
        for c in range(K // 16):
            col_v[j, pl.ds(c * 16, 16)] = padv
        return 0

    lax.fori_loop(nreal, nchunk, fill, 0)


def _fire_row(row_hbm, row_v, wid, nchunk, nreal, sem):
    base0 = wid * nchunk * K

    def issue(j, _):
        pltpu.async_copy(
            row_hbm.at[pl.ds(base0 + j * K, K)],
            row_v.at[pl.ds(j * K, K)], sem)
        return 0

    lax.fori_loop(0, nreal, issue, 0)

    padv = jnp.full((16,), N, jnp.int32)

    def fill(j, _):
        for c in range(K // 16):
            row_v[pl.ds(j * K + c * 16, 16)] = padv
        return 0

    lax.fori_loop(nreal, nchunk, fill, 0)


def _drain_idx(hbm, dst_slice, nreal, sem):
    def drain(j, _):
        pltpu.make_async_copy(hbm.at[pl.ds(0, K)], dst_slice, sem).wait()
        return 0

    lax.fori_loop(0, nreal, drain, 0)


def _make_deg_kernel(nchunk, e):
    @functools.partial(
        pl.kernel,
        out_type=jax.ShapeDtypeStruct((NC, N_PAD, DEGW), jnp.float32),
        mesh=_mesh,
        scratch_types=[
            pltpu.VMEM((nchunk, K), jnp.int32),
            pltpu.VMEM((K, DEGW), jnp.float32),
            pltpu.VMEM((RPT, DEGW), jnp.float32),
            pltpu.VMEM_SHARED((N_PAD, DEGW), jnp.float32),
            pltpu.SemaphoreType.DMA,
        ],
        compiler_params=pltpu.CompilerParams(use_tc_tiling_on_sc=False),
    )
    def deg_kernel(col_hbm, degp_hbm, col_v, ones_v, zrow_v, deg_sh, dsem):
        cid = lax.axis_index("c")
        sid = lax.axis_index("s")
        wid = sid * NC + cid
        nreal = _nreal(e, wid, nchunk)
        _fire_col(col_hbm, col_v, wid, nchunk, nreal, dsem)

        one = jnp.full((16,), 1.0, jnp.float32)

        def fill_ones(i, _):
            ones_v[i, :] = one
            return 0

        lax.fori_loop(0, K, fill_ones, 0)
        _zero_rows(zrow_v, RPT, DEGW)

        pltpu.sync_copy(zrow_v, deg_sh.at[pl.ds(sid * RPT, RPT)])
        _drain_idx(col_hbm, col_v.at[0], nreal, dsem)
        plsc.subcore_barrier()

        for j in range(8):
            pltpu.async_copy(ones_v, deg_sh.at[col_v.at[j]], dsem, add=True)

        def chunk(h, _):
            for j in range(8):
                s = 8 * h + j
                pltpu.make_async_copy(
                    ones_v, deg_sh.at[col_v.at[s]], dsem).wait()
                pltpu.async_copy(
                    ones_v, deg_sh.at[col_v.at[s + 8]], dsem, add=True)
            return 0

        lax.fori_loop(0, nchunk // 8 - 1, chunk, 0)
        for j in range(8):
            pltpu.make_async_copy(
                ones_v, deg_sh.at[col_v.at[j]], dsem).wait()
        plsc.subcore_barrier()

        pltpu.sync_copy(
            deg_sh.at[pl.ds(sid * RPT, RPT)],
            degp_hbm.at[cid, pl.ds(sid * RPT, RPT)],
        )

    return deg_kernel


def _ring(feat_sh, agg_sh, row_v, col_v, bufs, zrow_v, gsem, ssem, nchunk):
    def ridx(s):
        return row_v.at[pl.ds(s * K, K)]

    zsrc = zrow_v.at[pl.ds(0, K)]
    for j in range(HALF):
        pltpu.async_copy(
            zsrc, agg_sh.at[col_v.at[0]], ssem[HALF + j], add=True)
    for j in range(HALF):
        pltpu.async_copy(feat_sh.at[ridx(j)], bufs[j], gsem[j])

    def step(h, _):
        for j in range(NBUF):
            s = NBUF * h + j
            pltpu.make_async_copy(
                feat_sh.at[ridx(s)], bufs[j], gsem[j]).wait()
            pltpu.async_copy(
                bufs[j], agg_sh.at[col_v.at[s]], ssem[j], add=True)
            bb = (j + HALF) % NBUF
            pltpu.make_async_copy(
                bufs[bb], agg_sh.at[col_v.at[s]], ssem[bb]).wait()
            s2 = (s + HALF) % nchunk
            pltpu.async_copy(feat_sh.at[ridx(s2)], bufs[bb], gsem[bb])
        return 0

    lax.fori_loop(0, nchunk // NBUF, step, 0)

    for j in range(HALF):
        pltpu.make_async_copy(
            bufs[HALF + j], agg_sh.at[col_v.at[0]], ssem[HALF + j]).wait()
        pltpu.make_async_copy(
            feat_sh.at[ridx(j)], bufs[j], gsem[j]).wait()


def _make_agg1_kernel(nchunk, e):
    @functools.partial(
        pl.kernel,
        out_type=jax.ShapeDtypeStruct((NC, N_PAD, HID), jnp.float32),
        mesh=_mesh,
        scratch_types=[
            pltpu.VMEM((nchunk * K,), jnp.int32),
            pltpu.VMEM((nchunk, K), jnp.int32),
            [pltpu.VMEM((K, HID), jnp.float32) for _ in range(NBUF)],
            pltpu.VMEM((RPT, HID), jnp.float32),
            pltpu.VMEM_SHARED((N_PAD, HID), jnp.float32),
            pltpu.VMEM_SHARED((N_PAD, HID), jnp.float32),
            [pltpu.SemaphoreType.DMA for _ in range(NBUF)],
            [pltpu.SemaphoreType.DMA for _ in range(NBUF)],
        ],
        compiler_params=pltpu.CompilerParams(use_tc_tiling_on_sc=False),
    )
    def agg_kernel(feat_hbm, row_hbm, col_hbm, aggp_hbm,
                   row_v, col_v, bufs, zrow_v, agg_sh, feat_sh, gsem, ssem):
        cid = lax.axis_index("c")
        sid = lax.axis_index("s")
        wid = sid * NC + cid
        nreal = _nreal(e, wid, nchunk)

        _fire_row(row_hbm, row_v, wid, nchunk, nreal, gsem[0])
        _fire_col(col_hbm, col_v, wid, nchunk, nreal, gsem[1])

        pltpu.sync_copy(
            feat_hbm.at[pl.ds(sid * RPT, RPT)],
            feat_sh.at[pl.ds(sid * RPT, RPT)],
        )

        _zero_rows(zrow_v, RPT, HID)
        pltpu.sync_copy(zrow_v, agg_sh.at[pl.ds(sid * RPT, RPT)])
        _drain_idx(row_hbm, row_v.at[pl.ds(0, K)], nreal, gsem[0])
        _drain_idx(col_hbm, col_v.at[0], nreal, gsem[1])
        plsc.subcore_barrier()

        _ring(feat_sh, agg_sh, row_v, col_v, bufs, zrow_v, gsem, ssem, nchunk)
        plsc.subcore_barrier()

        pltpu.sync_copy(
            agg_sh.at[pl.ds(sid * RPT, RPT)],
            aggp_hbm.at[cid, pl.ds(sid * RPT, RPT)],
        )

    return agg_kernel


def _make_agg2_kernel(nchunk, e):
    @functools.partial(
        pl.kernel,
        out_type=jax.ShapeDtypeStruct((NC, N_PAD, HID), jnp.float32),
        mesh=_mesh,
        scratch_types=[
            pltpu.VMEM((nchunk * K,), jnp.int32),
            pltpu.VMEM((nchunk, K), jnp.int32),
            [pltpu.VMEM((K, HID), jnp.float32) for _ in range(NBUF)],
            pltpu.VMEM((RPT, HID), jnp.float32),
            pltpu.VMEM((HID,), jnp.float32),
            pltpu.VMEM_SHARED((N_PAD, HID), jnp.float32),
            pltpu.VMEM_SHARED((N_PAD, HID), jnp.float32),
            [pltpu.SemaphoreType.DMA for _ in range(NBUF)],
            [pltpu.SemaphoreType.DMA for _ in range(NBUF)],
        ],
        compiler_params=pltpu.CompilerParams(use_tc_tiling_on_sc=False),
    )
    def agg_kernel(aggp_in_hbm, dinvw_hbm, b1_hbm, row_hbm, col_hbm,
                   aggp_hbm, row_v, col_v, bufs, zrow_v, b1_v,
                   agg_sh, feat_sh, gsem, ssem):
        cid = lax.axis_index("c")
        sid = lax.axis_index("s")
        wid = sid * NC + cid
        nreal = _nreal(e, wid, nchunk)

        _fire_row(row_hbm, row_v, wid, nchunk, nreal, gsem[7])
        _fire_col(col_hbm, col_v, wid, nchunk, nreal, gsem[3])
        pltpu.sync_copy(b1_hbm, b1_v)

        def piece_srcs(p):
            base = sid * RPT + p * K
            return (aggp_in_hbm.at[0, pl.ds(base, K)],
                    aggp_in_hbm.at[1, pl.ds(base, K)],
                    dinvw_hbm.at[pl.ds(base, K)])

        def fire_piece(p):
            g = (p % 2) * 4
            for q, src in enumerate(piece_srcs(p)):
                pltpu.async_copy(src, bufs[g + q], gsem[g + q])

        fire_piece(0)
        fire_piece(1)
        _zero_rows(zrow_v, RPT, HID)

        for p in range(PIECES):
            g = (p % 2) * 4
            base = sid * RPT + p * K
            for q, src in enumerate(piece_srcs(p)):
                pltpu.make_async_copy(src, bufs[g + q], gsem[g + q]).wait()
            if p >= 2:
                pltpu.make_async_copy(
                    bufs[g + 3], feat_sh.at[pl.ds(0, K)],
                    ssem[p % 2]).wait()

            def row(i, _):
                for c in range(HID // 16):
                    sl = pl.ds(c * 16, 16)
                    d = bufs[g + 2][i, sl]
                    a = bufs[g][i, sl] + bufs[g + 1][i, sl]
                    h = jnp.maximum(a * d + b1_v[sl], 0.0)
                    bufs[g + 3][i, sl] = h * d
                return 0

            lax.fori_loop(0, K, row, 0)
            pltpu.async_copy(
                bufs[g + 3], feat_sh.at[pl.ds(base, K)], ssem[p % 2])
            if p + 2 < PIECES:
                fire_piece(p + 2)

        for p in (PIECES - 2, PIECES - 1):
            g = (p % 2) * 4
            pltpu.make_async_copy(
                bufs[g + 3], feat_sh.at[pl.ds(0, K)], ssem[p % 2]).wait()

        pltpu.sync_copy(zrow_v, agg_sh.at[pl.ds(sid * RPT, RPT)])
        _drain_idx(row_hbm, row_v.at[pl.ds(0, K)], nreal, gsem[7])
        _drain_idx(col_hbm, col_v.at[0], nreal, gsem[3])
        plsc.subcore_barrier()

        _ring(feat_sh, agg_sh, row_v, col_v, bufs, zrow_v, gsem, ssem, nchunk)
        plsc.subcore_barrier()

        pltpu.sync_copy(
            agg_sh.at[pl.ds(sid * RPT, RPT)],
            aggp_hbm.at[cid, pl.ds(sid * RPT, RPT)],
        )

    return agg_kernel


def _tc_xw_scale(x, w1, degp):
    def body(x_ref, w_ref, degp_ref, t1_ref, dinv_ref):
        deg = degp_ref[0, :, 0:1] + degp_ref[1, :, 0:1]
        rows = lax.broadcasted_iota(jnp.int32, (N_PAD, 1), 0)
        dinv = jnp.where(
            (deg > 0) & (rows < N),
            lax.rsqrt(jnp.maximum(deg, 1e-12)), 0.0)
        xw = jnp.dot(x_ref[...], w_ref[...],
                     preferred_element_type=jnp.float32)
        xw_pad = jnp.concatenate(
            [xw, jnp.zeros((N_PAD - N, HID), jnp.float32)], axis=0)
        t1_ref[...] = xw_pad * dinv
        dinv_ref[...] = jnp.broadcast_to(dinv, (N_PAD, HID))

    return pl.pallas_call(
        body,
        out_shape=(
            jax.ShapeDtypeStruct((N_PAD, HID), jnp.float32),
            jax.ShapeDtypeStruct((N_PAD, HID), jnp.float32),
        ),
    )(x, w1, degp)


def _tc_final(aggp_f, dinvw_f, w2, b2):
    def body(aggp_ref, dinv_ref, w2_ref, b2_ref, out_ref):
        agg = aggp_ref[0] + aggp_ref[1]
        dv = dinv_ref[...]
        y = jnp.dot((agg * dv)[:N, :], w2_ref[...],
                    preferred_element_type=jnp.float32) + b2_ref[...]
        m = jnp.max(y, axis=1, keepdims=True)
        s = y - m
        lse = jnp.log(jnp.sum(jnp.exp(s), axis=1, keepdims=True))
        out_ref[...] = s - lse

    return pl.pallas_call(
        body,
        out_shape=jax.ShapeDtypeStruct((N, 128), jnp.float32),
    )(aggp_f, dinvw_f, w2, b2)


def kernel(x, edge_index, W1, b1, W2, b2):
    e = edge_index.shape[1]
    nchunk = -(-e // (NW * K))
    nchunk = -(-nchunk // NBUF) * NBUF

    row = edge_index[0].astype(jnp.int32)
    col = edge_index[1].astype(jnp.int32)
    if e % K:
        pad = jnp.full((K - e % K,), N, jnp.int32)
        row = jnp.concatenate([row, pad])
        col = jnp.concatenate([col, pad])
        e = row.shape[0]

    degp = _make_deg_kernel(nchunk, e)(col)
    t1s, dinvw = _tc_xw_scale(x, W1, degp)
    agg1p = _make_agg1_kernel(nchunk, e)(t1s, row, col)
    agg2p = _make_agg2_kernel(nchunk, e)(agg1p, dinvw, b1, row, col)
    return _tc_final(agg2p, dinvw, W2, b2)


# speedup vs baseline: 1.1967x; 1.0557x over previous
import functools

import jax
import jax.numpy as jnp
from jax import lax
from jax.experimental import pallas as pl
from jax.experimental.pallas import tpu as pltpu
from jax.experimental.pallas import tpu_sc as plsc

N = 10000
HID = 32
DEGW = 16

NC = 2
NS = 16
NW = NC * NS
K = 128
NBUF = 10
HALF = NBUF // 2

N_PAD = 10240
RPT = N_PAD // NS
FN = N_PAD * HID // 128
PIECES = RPT // K

_mesh = plsc.VectorSubcoreMesh(core_axis_name="c", subcore_axis_name="s")


def _zero_rows(ref, nrows, ncols):
    zero = jnp.zeros((16,), jnp.float32)

    def body(i, _):
        for c in range(ncols // 16):
            ref[i, pl.ds(c * 16, 16)] = zero
        return 0

    lax.fori_loop(0, nrows, body, 0)


def _nreal(e, wid, nchunk):
    return jnp.clip(e // K - wid * nchunk, 0, nchunk)


def _fire_col(ei_hbm, col_v, wid, nchunk, nreal, sem):
    base0 = wid * nchunk * K

    def issue(j, _):
        pltpu.async_copy(
            ei_hbm.at[1, pl.ds(base0 + j * K, K)], col_v.at[j], sem)
        return 0

    lax.fori_loop(0, nreal, issue, 0)

    padv = jnp.full((16,), N, jnp.int32)

    def fill(j, _):
        for c in range(K // 16):
            col_v[j, pl.ds(c * 16, 16)] = padv
        return 0

    lax.fori_loop(nreal, nchunk, fill, 0)


def _fire_row(ei_hbm, row_v, wid, nchunk, nreal, sem):
    base0 = wid * nchunk * K

    def issue(j, _):
        pltpu.async_copy(
            ei_hbm.at[0, pl.ds(base0 + j * K, K)],
            row_v.at[pl.ds(j * K, K)], sem)
        return 0

    lax.fori_loop(0, nreal, issue, 0)

    padv = jnp.full((16,), N, jnp.int32)

    def fill(j, _):
        for c in range(K // 16):
            row_v[pl.ds(j * K + c * 16, 16)] = padv
        return 0

    lax.fori_loop(nreal, nchunk, fill, 0)


def _drain_idx(ei_hbm, dst_slice, nreal, sem):
    def drain(j, _):
        pltpu.make_async_copy(
            ei_hbm.at[0, pl.ds(0, K)], dst_slice, sem).wait()
        return 0

    lax.fori_loop(0, nreal, drain, 0)


def _make_deg_kernel(nchunk, e):
    @functools.partial(
        pl.kernel,
        out_type=jax.ShapeDtypeStruct((NC, N_PAD, DEGW), jnp.float32),
        mesh=_mesh,
        scratch_types=[
            pltpu.VMEM((nchunk, K), jnp.int32),
            pltpu.VMEM((K, DEGW), jnp.float32),
            pltpu.VMEM((RPT, DEGW), jnp.float32),
            pltpu.VMEM_SHARED((N_PAD, DEGW), jnp.float32),
            pltpu.SemaphoreType.DMA,
        ],
        compiler_params=pltpu.CompilerParams(use_tc_tiling_on_sc=False),
    )
    def deg_kernel(ei_hbm, degp_hbm, col_v, ones_v, zrow_v, deg_sh, dsem):
        cid = lax.axis_index("c")
        sid = lax.axis_index("s")
        wid = sid * NC + cid
        nreal = _nreal(e, wid, nchunk)
        _fire_col(ei_hbm, col_v, wid, nchunk, nreal, dsem)

        one = jnp.full((16,), 1.0, jnp.float32)

        def fill_ones(i, _):
            ones_v[i, :] = one
            return 0

        lax.fori_loop(0, K, fill_ones, 0)
        _zero_rows(zrow_v, RPT, DEGW)

        pltpu.sync_copy(zrow_v, deg_sh.at[pl.ds(sid * RPT, RPT)])
        _drain_idx(ei_hbm, col_v.at[0], nreal, dsem)
        plsc.subcore_barrier()

        for j in range(8):
            pltpu.async_copy(ones_v, deg_sh.at[col_v.at[j]], dsem, add=True)

        def chunk(h, _):
            for j in range(8):
                s = 8 * h + j
                pltpu.make_async_copy(
                    ones_v, deg_sh.at[col_v.at[s]], dsem).wait()
                pltpu.async_copy(
                    ones_v, deg_sh.at[col_v.at[s + 8]], dsem, add=True)
            return 0

        lax.fori_loop(0, nchunk // 8 - 1, chunk, 0)
        for j in range(8):
            pltpu.make_async_copy(
                ones_v, deg_sh.at[col_v.at[j]], dsem).wait()
        plsc.subcore_barrier()

        pltpu.sync_copy(
            deg_sh.at[pl.ds(sid * RPT, RPT)],
            degp_hbm.at[cid, pl.ds(sid * RPT, RPT)],
        )

    return deg_kernel


def _ring(feat_sh, agg_sh, row_v, col_v, bufs, zrow_v, gsem, ssem, nchunk):
    def ridx(s):
        return row_v.at[pl.ds(s * K, K)]

    zsrc = zrow_v.at[pl.ds(0, K)]
    for j in range(HALF):
        pltpu.async_copy(
            zsrc, agg_sh.at[col_v.at[0]], ssem[HALF + j], add=True)
    for j in range(HALF):
        pltpu.async_copy(feat_sh.at[ridx(j)], bufs[j], gsem[j])

    def step(h, _):
        for j in range(NBUF):
            s = NBUF * h + j
            pltpu.make_async_copy(
                feat_sh.at[ridx(s)], bufs[j], gsem[j]).wait()
            pltpu.async_copy(
                bufs[j], agg_sh.at[col_v.at[s]], ssem[j], add=True)
            bb = (j + HALF) % NBUF
            pltpu.make_async_copy(
                bufs[bb], agg_sh.at[col_v.at[s]], ssem[bb]).wait()
            s2 = (s + HALF) % nchunk
            pltpu.async_copy(feat_sh.at[ridx(s2)], bufs[bb], gsem[bb])
        return 0

    lax.fori_loop(0, nchunk // NBUF, step, 0)

    for j in range(HALF):
        pltpu.make_async_copy(
            bufs[HALF + j], agg_sh.at[col_v.at[0]], ssem[HALF + j]).wait()
        pltpu.make_async_copy(
            feat_sh.at[ridx(j)], bufs[j], gsem[j]).wait()


def _make_agg1_kernel(nchunk, e):
    @functools.partial(
        pl.kernel,
        out_type=jax.ShapeDtypeStruct((NC, N_PAD, HID), jnp.float32),
        mesh=_mesh,
        scratch_types=[
            pltpu.VMEM((nchunk * K,), jnp.int32),
            pltpu.VMEM((nchunk, K), jnp.int32),
            [pltpu.VMEM((K, HID), jnp.float32) for _ in range(NBUF)],
            pltpu.VMEM((RPT, HID), jnp.float32),
            pltpu.VMEM_SHARED((N_PAD, HID), jnp.float32),
            pltpu.VMEM_SHARED((N_PAD, HID), jnp.float32),
            [pltpu.SemaphoreType.DMA for _ in range(NBUF)],
            [pltpu.SemaphoreType.DMA for _ in range(NBUF)],
        ],
        compiler_params=pltpu.CompilerParams(use_tc_tiling_on_sc=False),
    )
    def agg_kernel(feat_hbm, ei_hbm, aggp_hbm,
                   row_v, col_v, bufs, zrow_v, agg_sh, feat_sh, gsem, ssem):
        cid = lax.axis_index("c")
        sid = lax.axis_index("s")
        wid = sid * NC + cid
        nreal = _nreal(e, wid, nchunk)

        _fire_row(ei_hbm, row_v, wid, nchunk, nreal, gsem[0])
        _fire_col(ei_hbm, col_v, wid, nchunk, nreal, gsem[1])

        pltpu.sync_copy(
            feat_hbm.at[pl.ds(sid * RPT, RPT)],
            feat_sh.at[pl.ds(sid * RPT, RPT)],
        )

        _zero_rows(zrow_v, RPT, HID)
        pltpu.sync_copy(zrow_v, agg_sh.at[pl.ds(sid * RPT, RPT)])
        _drain_idx(ei_hbm, row_v.at[pl.ds(0, K)], nreal, gsem[0])
        _drain_idx(ei_hbm, col_v.at[0], nreal, gsem[1])
        plsc.subcore_barrier()

        _ring(feat_sh, agg_sh, row_v, col_v, bufs, zrow_v, gsem, ssem, nchunk)
        plsc.subcore_barrier()

        pltpu.sync_copy(
            agg_sh.at[pl.ds(sid * RPT, RPT)],
            aggp_hbm.at[cid, pl.ds(sid * RPT, RPT)],
        )

    return agg_kernel


def _make_agg2_kernel(nchunk, e):
    @functools.partial(
        pl.kernel,
        out_type=jax.ShapeDtypeStruct((NC, N_PAD, HID), jnp.float32),
        mesh=_mesh,
        scratch_types=[
            pltpu.VMEM((nchunk * K,), jnp.int32),
            pltpu.VMEM((nchunk, K), jnp.int32),
            [pltpu.VMEM((K, HID), jnp.float32) for _ in range(NBUF)],
            pltpu.VMEM((RPT, HID), jnp.float32),
            pltpu.VMEM((HID,), jnp.float32),
            pltpu.VMEM_SHARED((N_PAD, HID), jnp.float32),
            pltpu.VMEM_SHARED((N_PAD, HID), jnp.float32),
            [pltpu.SemaphoreType.DMA for _ in range(NBUF)],
            [pltpu.SemaphoreType.DMA for _ in range(NBUF)],
        ],
        compiler_params=pltpu.CompilerParams(use_tc_tiling_on_sc=False),
    )
    def agg_kernel(aggp_in_hbm, dinvw_hbm, b1_hbm, ei_hbm,
                   aggp_hbm, row_v, col_v, bufs, zrow_v, b1_v,
                   agg_sh, feat_sh, gsem, ssem):
        cid = lax.axis_index("c")
        sid = lax.axis_index("s")
        wid = sid * NC + cid
        nreal = _nreal(e, wid, nchunk)

        _fire_row(ei_hbm, row_v, wid, nchunk, nreal, gsem[7])
        _fire_col(ei_hbm, col_v, wid, nchunk, nreal, gsem[3])
        pltpu.sync_copy(b1_hbm, b1_v)

        def piece_srcs(p):
            base = sid * RPT + p * K
            return (aggp_in_hbm.at[0, pl.ds(base, K)],
                    aggp_in_hbm.at[1, pl.ds(base, K)],
                    dinvw_hbm.at[pl.ds(base, K)])

        def fire_piece(p):
            g = (p % 2) * 4
            for q, src in enumerate(piece_srcs(p)):
                pltpu.async_copy(src, bufs[g + q], gsem[g + q])

        fire_piece(0)
        fire_piece(1)
        _zero_rows(zrow_v, RPT, HID)

        for p in range(PIECES):
            g = (p % 2) * 4
            base = sid * RPT + p * K
            for q, src in enumerate(piece_srcs(p)):
                pltpu.make_async_copy(src, bufs[g + q], gsem[g + q]).wait()
            if p >= 2:
                pltpu.make_async_copy(
                    bufs[g + 3], feat_sh.at[pl.ds(0, K)],
                    ssem[p % 2]).wait()

            def row(i, _):
                for c in range(HID // 16):
                    sl = pl.ds(c * 16, 16)
                    d = bufs[g + 2][i, sl]
                    a = bufs[g][i, sl] + bufs[g + 1][i, sl]
                    h = jnp.maximum(a * d + b1_v[sl], 0.0)
                    bufs[g + 3][i, sl] = h * d
                return 0

            lax.fori_loop(0, K, row, 0)
            pltpu.async_copy(
                bufs[g + 3], feat_sh.at[pl.ds(base, K)], ssem[p % 2])
            if p + 2 < PIECES:
                fire_piece(p + 2)

        for p in (PIECES - 2, PIECES - 1):
            g = (p % 2) * 4
            pltpu.make_async_copy(
                bufs[g + 3], feat_sh.at[pl.ds(0, K)], ssem[p % 2]).wait()

        pltpu.sync_copy(zrow_v, agg_sh.at[pl.ds(sid * RPT, RPT)])
        _drain_idx(ei_hbm, row_v.at[pl.ds(0, K)], nreal, gsem[7])
        _drain_idx(ei_hbm, col_v.at[0], nreal, gsem[3])
        plsc.subcore_barrier()

        _ring(feat_sh, agg_sh, row_v, col_v, bufs, zrow_v, gsem, ssem, nchunk)
        plsc.subcore_barrier()

        pltpu.sync_copy(
            agg_sh.at[pl.ds(sid * RPT, RPT)],
            aggp_hbm.at[cid, pl.ds(sid * RPT, RPT)],
        )

    return agg_kernel


def _tc_xw_scale(x, w1, degp):
    def body(x_ref, w_ref, degp_ref, t1_ref, dinv_ref):
        deg = degp_ref[0, :, 0:1] + degp_ref[1, :, 0:1]
        rows = lax.broadcasted_iota(jnp.int32, (N_PAD, 1), 0)
        dinv = jnp.where(
            (deg > 0) & (rows < N),
            lax.rsqrt(jnp.maximum(deg, 1e-12)), 0.0)
        xw = jnp.dot(x_ref[...], w_ref[...],
                     preferred_element_type=jnp.float32)
        xw_pad = jnp.concatenate(
            [xw, jnp.zeros((N_PAD - N, HID), jnp.float32)], axis=0)
        t1_ref[...] = xw_pad * dinv
        dinv_ref[...] = jnp.broadcast_to(dinv, (N_PAD, HID))

    return pl.pallas_call(
        body,
        out_shape=(
            jax.ShapeDtypeStruct((N_PAD, HID), jnp.float32),
            jax.ShapeDtypeStruct((N_PAD, HID), jnp.float32),
        ),
    )(x, w1, degp)


def _tc_final(aggp_f, dinvw_f, w2, b2):
    def body(aggp_ref, dinv_ref, w2_ref, b2_ref, out_ref):
        agg = aggp_ref[0] + aggp_ref[1]
        dv = dinv_ref[...]
        y = jnp.dot((agg * dv)[:N, :], w2_ref[...],
                    preferred_element_type=jnp.float32) + b2_ref[...]
        m = jnp.max(y, axis=1, keepdims=True)
        s = y - m
        lse = jnp.log(jnp.sum(jnp.exp(s), axis=1, keepdims=True))
        out_ref[...] = s - lse

    return pl.pallas_call(
        body,
        out_shape=jax.ShapeDtypeStruct((N, 128), jnp.float32),
    )(aggp_f, dinvw_f, w2, b2)


def kernel(x, edge_index, W1, b1, W2, b2):
    e = edge_index.shape[1]
    nchunk = -(-e // (NW * K))
    nchunk = -(-nchunk // NBUF) * NBUF

    ei = edge_index.astype(jnp.int32)
    if e % K:
        pad = jnp.full((2, K - e % K), N, jnp.int32)
        ei = jnp.concatenate([ei, pad], axis=1)
        e = ei.shape[1]

    degp = _make_deg_kernel(nchunk, e)(ei)
    t1s, dinvw = _tc_xw_scale(x, W1, degp)
    agg1p = _make_agg1_kernel(nchunk, e)(t1s, ei)
    agg2p = _make_agg2_kernel(nchunk, e)(agg1p, dinvw, b1, ei)
    return _tc_final(agg2p, dinvw, W2, b2)


# speedup vs baseline: 1.2436x; 1.0392x over previous
import functools

import jax
import jax.numpy as jnp
from jax import lax
from jax.experimental import pallas as pl
from jax.experimental.pallas import tpu as pltpu
from jax.experimental.pallas import tpu_sc as plsc

N = 10000
HID = 32
DEGW = 16

NC = 2
NS = 16
NW = NC * NS
K = 128
NBUF = 10
HALF = NBUF // 2

N_PAD = 10240
RPT = N_PAD // NS
FN = N_PAD * HID // 128
PIECES = RPT // K

_mesh = plsc.VectorSubcoreMesh(core_axis_name="c", subcore_axis_name="s")


def _zero_rows(ref, nrows, ncols):
    zero = jnp.zeros((16,), jnp.float32)

    def body(i, _):
        for c in range(ncols // 16):
            ref[i, pl.ds(c * 16, 16)] = zero
        return 0

    lax.fori_loop(0, nrows, body, 0)


def _nreal(e, wid, nchunk):
    return jnp.clip(e // K - wid * nchunk, 0, nchunk)


def _fire_col(ei_hbm, col_v, wid, nchunk, nreal, sem):
    base0 = wid * nchunk * K

    def issue(j, _):
        pltpu.async_copy(
            ei_hbm.at[1, pl.ds(base0 + j * K, K)], col_v.at[j], sem)
        return 0

    lax.fori_loop(0, nreal, issue, 0)

    padv = jnp.full((16,), N, jnp.int32)

    def fill(j, _):
        for c in range(K // 16):
            col_v[j, pl.ds(c * 16, 16)] = padv
        return 0

    lax.fori_loop(nreal, nchunk, fill, 0)


def _fire_row(ei_hbm, row_v, wid, nchunk, nreal, sem):
    base0 = wid * nchunk * K

    def issue(j, _):
        pltpu.async_copy(
            ei_hbm.at[0, pl.ds(base0 + j * K, K)],
            row_v.at[pl.ds(j * K, K)], sem)
        return 0

    lax.fori_loop(0, nreal, issue, 0)

    padv = jnp.full((16,), N, jnp.int32)

    def fill(j, _):
        for c in range(K // 16):
            row_v[pl.ds(j * K + c * 16, 16)] = padv
        return 0

    lax.fori_loop(nreal, nchunk, fill, 0)


def _drain_idx(ei_hbm, dst_slice, nreal, sem):
    def drain(j, _):
        pltpu.make_async_copy(
            ei_hbm.at[0, pl.ds(0, K)], dst_slice, sem).wait()
        return 0

    lax.fori_loop(0, nreal, drain, 0)


def _make_deg_kernel(nchunk, e):
    @functools.partial(
        pl.kernel,
        out_type=jax.ShapeDtypeStruct((NC, N_PAD, DEGW), jnp.float32),
        mesh=_mesh,
        scratch_types=[
            pltpu.VMEM((nchunk, K), jnp.int32),
            pltpu.VMEM((K, DEGW), jnp.float32),
            pltpu.VMEM((RPT, DEGW), jnp.float32),
            pltpu.VMEM_SHARED((N_PAD, DEGW), jnp.float32),
            pltpu.SemaphoreType.DMA,
        ],
        compiler_params=pltpu.CompilerParams(use_tc_tiling_on_sc=False),
    )
    def deg_kernel(ei_hbm, degp_hbm, col_v, ones_v, zrow_v, deg_sh, dsem):
        cid = lax.axis_index("c")
        sid = lax.axis_index("s")
        wid = sid * NC + cid
        nreal = _nreal(e, wid, nchunk)
        _fire_col(ei_hbm, col_v, wid, nchunk, nreal, dsem)

        one = jnp.full((16,), 1.0, jnp.float32)

        def fill_ones(i, _):
            ones_v[i, :] = one
            return 0

        lax.fori_loop(0, K, fill_ones, 0)
        _zero_rows(zrow_v, RPT, DEGW)

        pltpu.sync_copy(zrow_v, deg_sh.at[pl.ds(sid * RPT, RPT)])
        _drain_idx(ei_hbm, col_v.at[0], nreal, dsem)
        plsc.subcore_barrier()

        for j in range(8):
            pltpu.async_copy(ones_v, deg_sh.at[col_v.at[j]], dsem, add=True)

        def chunk(h, _):
            for j in range(8):
                s = 8 * h + j
                pltpu.make_async_copy(
                    ones_v, deg_sh.at[col_v.at[s]], dsem).wait()
                pltpu.async_copy(
                    ones_v, deg_sh.at[col_v.at[s + 8]], dsem, add=True)
            return 0

        lax.fori_loop(0, nchunk // 8 - 1, chunk, 0)
        for j in range(8):
            pltpu.make_async_copy(
                ones_v, deg_sh.at[col_v.at[j]], dsem).wait()
        plsc.subcore_barrier()

        pltpu.sync_copy(
            deg_sh.at[pl.ds(sid * RPT, RPT)],
            degp_hbm.at[cid, pl.ds(sid * RPT, RPT)],
        )

    return deg_kernel


def _ring(feat_sh, agg_sh, row_v, col_v, bufs, zrow_v, gsem, ssem, nchunk):
    def ridx(s):
        return row_v.at[pl.ds(s * K, K)]

    zsrc = zrow_v.at[pl.ds(0, K)]
    for j in range(HALF):
        pltpu.async_copy(
            zsrc, agg_sh.at[col_v.at[0]], ssem[HALF + j], add=True)
    for j in range(HALF):
        pltpu.async_copy(feat_sh.at[ridx(j)], bufs[j], gsem[j])

    def step(h, _):
        for j in range(NBUF):
            s = NBUF * h + j
            pltpu.make_async_copy(
                feat_sh.at[ridx(s)], bufs[j], gsem[j]).wait()
            pltpu.async_copy(
                bufs[j], agg_sh.at[col_v.at[s]], ssem[j], add=True)
            bb = (j + HALF) % NBUF
            pltpu.make_async_copy(
                bufs[bb], agg_sh.at[col_v.at[s]], ssem[bb]).wait()
            s2 = (s + HALF) % nchunk
            pltpu.async_copy(feat_sh.at[ridx(s2)], bufs[bb], gsem[bb])
        return 0

    lax.fori_loop(0, nchunk // NBUF, step, 0)

    for j in range(HALF):
        pltpu.make_async_copy(
            bufs[HALF + j], agg_sh.at[col_v.at[0]], ssem[HALF + j]).wait()
        pltpu.make_async_copy(
            feat_sh.at[ridx(j)], bufs[j], gsem[j]).wait()


def _make_agg1_kernel(nchunk, e):
    @functools.partial(
        pl.kernel,
        out_type=jax.ShapeDtypeStruct((NC, N_PAD, HID), jnp.float32),
        mesh=_mesh,
        scratch_types=[
            pltpu.VMEM((nchunk * K,), jnp.int32),
            pltpu.VMEM((nchunk, K), jnp.int32),
            [pltpu.VMEM((K, HID), jnp.float32) for _ in range(NBUF)],
            pltpu.VMEM((RPT, HID), jnp.float32),
            pltpu.VMEM_SHARED((N_PAD, HID), jnp.float32),
            pltpu.VMEM_SHARED((N_PAD, HID), jnp.float32),
            [pltpu.SemaphoreType.DMA for _ in range(NBUF)],
            [pltpu.SemaphoreType.DMA for _ in range(NBUF)],
        ],
        compiler_params=pltpu.CompilerParams(use_tc_tiling_on_sc=False),
    )
    def agg_kernel(feat_hbm, ei_hbm, aggp_hbm,
                   row_v, col_v, bufs, zrow_v, agg_sh, feat_sh, gsem, ssem):
        cid = lax.axis_index("c")
        sid = lax.axis_index("s")
        wid = sid * NC + cid
        nreal = _nreal(e, wid, nchunk)

        _fire_row(ei_hbm, row_v, wid, nchunk, nreal, gsem[0])
        _fire_col(ei_hbm, col_v, wid, nchunk, nreal, gsem[1])

        pltpu.sync_copy(
            feat_hbm.at[pl.ds(sid * RPT, RPT)],
            feat_sh.at[pl.ds(sid * RPT, RPT)],
        )

        _zero_rows(zrow_v, RPT, HID)
        pltpu.sync_copy(zrow_v, agg_sh.at[pl.ds(sid * RPT, RPT)])
        _drain_idx(ei_hbm, row_v.at[pl.ds(0, K)], nreal, gsem[0])
        _drain_idx(ei_hbm, col_v.at[0], nreal, gsem[1])
        plsc.subcore_barrier()

        _ring(feat_sh, agg_sh, row_v, col_v, bufs, zrow_v, gsem, ssem, nchunk)
        plsc.subcore_barrier()

        pltpu.sync_copy(
            agg_sh.at[pl.ds(sid * RPT, RPT)],
            aggp_hbm.at[cid, pl.ds(sid * RPT, RPT)],
        )

    return agg_kernel


def _make_agg2_kernel(nchunk, e):
    @functools.partial(
        pl.kernel,
        out_type=jax.ShapeDtypeStruct((NC, N_PAD, HID), jnp.float32),
        mesh=_mesh,
        scratch_types=[
            pltpu.VMEM((nchunk * K,), jnp.int32),
            pltpu.VMEM((nchunk, K), jnp.int32),
            [pltpu.VMEM((K, HID), jnp.float32) for _ in range(NBUF)],
            pltpu.VMEM((RPT, HID), jnp.float32),
            pltpu.VMEM((HID,), jnp.float32),
            pltpu.VMEM_SHARED((N_PAD, HID), jnp.float32),
            pltpu.VMEM_SHARED((N_PAD, HID), jnp.float32),
            [pltpu.SemaphoreType.DMA for _ in range(NBUF)],
            [pltpu.SemaphoreType.DMA for _ in range(NBUF)],
        ],
        compiler_params=pltpu.CompilerParams(use_tc_tiling_on_sc=False),
    )
    def agg_kernel(aggp_in_hbm, dinvw_hbm, b1_hbm, ei_hbm,
                   aggp_hbm, row_v, col_v, bufs, zrow_v, b1_v,
                   agg_sh, feat_sh, gsem, ssem):
        cid = lax.axis_index("c")
        sid = lax.axis_index("s")
        wid = sid * NC + cid
        nreal = _nreal(e, wid, nchunk)

        _fire_row(ei_hbm, row_v, wid, nchunk, nreal, gsem[7])
        _fire_col(ei_hbm, col_v, wid, nchunk, nreal, gsem[3])
        pltpu.sync_copy(b1_hbm, b1_v)

        def piece_srcs(p):
            base = sid * RPT + p * K
            return (aggp_in_hbm.at[0, pl.ds(base, K)],
                    aggp_in_hbm.at[1, pl.ds(base, K)],
                    dinvw_hbm.at[pl.ds(base, K)])

        def fire_piece(p):
            g = (p % 2) * 4
            for q, src in enumerate(piece_srcs(p)):
                pltpu.async_copy(src, bufs[g + q], gsem[g + q])

        fire_piece(0)
        fire_piece(1)
        _zero_rows(zrow_v, RPT, HID)

        for p in range(PIECES):
            g = (p % 2) * 4
            base = sid * RPT + p * K
            for q, src in enumerate(piece_srcs(p)):
                pltpu.make_async_copy(src, bufs[g + q], gsem[g + q]).wait()
            if p >= 2:
                pltpu.make_async_copy(
                    bufs[g + 3], feat_sh.at[pl.ds(0, K)],
                    ssem[p % 2]).wait()

            def row(i, _):
                for c in range(HID // 16):
                    sl = pl.ds(c * 16, 16)
                    d = bufs[g + 2][i, sl]
                    a = bufs[g][i, sl] + bufs[g + 1][i, sl]
                    h = jnp.maximum(a * d + b1_v[sl], 0.0)
                    bufs[g + 3][i, sl] = h * d
                return 0

            lax.fori_loop(0, K, row, 0)
            pltpu.async_copy(
                bufs[g + 3], feat_sh.at[pl.ds(base, K)], ssem[p % 2])
            if p + 2 < PIECES:
                fire_piece(p + 2)

        for p in (PIECES - 2, PIECES - 1):
            g = (p % 2) * 4
            pltpu.make_async_copy(
                bufs[g + 3], feat_sh.at[pl.ds(0, K)], ssem[p % 2]).wait()

        pltpu.sync_copy(zrow_v, agg_sh.at[pl.ds(sid * RPT, RPT)])
        _drain_idx(ei_hbm, row_v.at[pl.ds(0, K)], nreal, gsem[7])
        _drain_idx(ei_hbm, col_v.at[0], nreal, gsem[3])
        plsc.subcore_barrier()

        _ring(feat_sh, agg_sh, row_v, col_v, bufs, zrow_v, gsem, ssem, nchunk)
        plsc.subcore_barrier()

        pltpu.sync_copy(
            agg_sh.at[pl.ds(sid * RPT, RPT)],
            aggp_hbm.at[cid, pl.ds(sid * RPT, RPT)],
        )

    return agg_kernel


def _tc_xw_scale(x, w1, degp):
    def body(x_ref, w_ref, degp_ref, t1_ref, dinv_ref):
        deg = degp_ref[0, :, 0:1] + degp_ref[1, :, 0:1]
        rows = lax.broadcasted_iota(jnp.int32, (N_PAD, 1), 0)
        dinv = jnp.where(
            (deg > 0) & (rows < N),
            lax.rsqrt(jnp.maximum(deg, 1e-12)), 0.0)
        w_wide = jnp.concatenate([w_ref[...]] * 4, axis=1)
        xw = jnp.dot(x_ref[...], w_wide,
                     preferred_element_type=jnp.float32)
        xww = jnp.concatenate(
            [xw, jnp.zeros((N_PAD - N, 128), jnp.float32)], axis=0) * dinv
        lane = lax.broadcasted_iota(jnp.int32, (1, 128), 1)
        x3 = xww.reshape(FN, 4, 128)
        dv3 = jnp.broadcast_to(dinv, (N_PAD, 128)).reshape(FN, 4, 128)
        t1 = jnp.zeros((FN, 128), jnp.float32)
        dw = jnp.zeros((FN, 128), jnp.float32)
        for c in range(4):
            lm = ((lane >= 32 * c) & (lane < 32 * (c + 1))).astype(
                jnp.float32)
            t1 = t1 + x3[:, c, :] * lm
            dw = dw + dv3[:, c, :] * lm
        t1_ref[...] = t1
        dinv_ref[...] = dw

    return pl.pallas_call(
        body,
        out_shape=(
            jax.ShapeDtypeStruct((FN, 128), jnp.float32),
            jax.ShapeDtypeStruct((FN, 128), jnp.float32),
        ),
    )(x, w1, degp)


def _tc_final(aggp_f, dinvw_f, w2, b2):
    def body(aggp_ref, dinv_ref, w2_ref, b2_ref, out_ref):
        z = (aggp_ref[0] + aggp_ref[1]) * dinv_ref[...]
        rep = jnp.broadcast_to(
            z.reshape(FN, 1, 128), (FN, 4, 128)).reshape(N_PAD, 128)
        phase = lax.broadcasted_iota(jnp.int32, (N_PAD, 1), 0) % 4
        yin = jnp.zeros((N_PAD, HID), jnp.float32)
        for c in range(4):
            yin = yin + jnp.where(
                phase == c, rep[:, 32 * c:32 * (c + 1)], 0.0)
        y = jnp.dot(yin[:N, :], w2_ref[...],
                    preferred_element_type=jnp.float32) + b2_ref[...]
        m = jnp.max(y, axis=1, keepdims=True)
        s = y - m
        lse = jnp.log(jnp.sum(jnp.exp(s), axis=1, keepdims=True))
        out_ref[...] = s - lse

    return pl.pallas_call(
        body,
        out_shape=jax.ShapeDtypeStruct((N, 128), jnp.float32),
    )(aggp_f, dinvw_f, w2, b2)


def kernel(x, edge_index, W1, b1, W2, b2):
    e = edge_index.shape[1]
    nchunk = -(-e // (NW * K))
    nchunk = -(-nchunk // NBUF) * NBUF

    ei = edge_index.astype(jnp.int32)
    if e % K:
        pad = jnp.full((2, K - e % K), N, jnp.int32)
        ei = jnp.concatenate([ei, pad], axis=1)
        e = ei.shape[1]

    degp = _make_deg_kernel(nchunk, e)(ei)
    t1s_f, dinvw_f = _tc_xw_scale(x, W1, degp)
    agg1p = _make_agg1_kernel(nchunk, e)(t1s_f.reshape(N_PAD, HID), ei)
    agg2p = _make_agg2_kernel(nchunk, e)(
        agg1p, dinvw_f.reshape(N_PAD, HID), b1, ei)
    return _tc_final(agg2p.reshape(NC, FN, 128), dinvw_f, W2, b2)


# speedup vs baseline: 1.2542x; 1.0085x over previous
import functools

import jax
import jax.numpy as jnp
from jax import lax
from jax.experimental import pallas as pl
from jax.experimental.pallas import tpu as pltpu
from jax.experimental.pallas import tpu_sc as plsc

N = 10000
HID = 32
DEGW = 16

NC = 2
NS = 16
NW = NC * NS
K = 128
NBUF = 10
HALF = NBUF // 2

N_PAD = 10240
RPT = N_PAD // NS
FN = N_PAD * HID // 128
PIECES = RPT // K

_mesh = plsc.VectorSubcoreMesh(core_axis_name="c", subcore_axis_name="s")


def _zero_rows(ref, nrows, ncols):
    zero = jnp.zeros((16,), jnp.float32)

    def body(i, _):
        for c in range(ncols // 16):
            ref[i, pl.ds(c * 16, 16)] = zero
        return 0

    lax.fori_loop(0, nrows, body, 0)


def _nreal(e, wid, nchunk):
    return jnp.clip(e // K - wid * nchunk, 0, nchunk)


def _fire_col(ei_hbm, col_v, wid, nchunk, nreal, sem):
    base0 = wid * nchunk * K

    def issue(j, _):
        pltpu.async_copy(
            ei_hbm.at[1, pl.ds(base0 + j * K, K)], col_v.at[j], sem)
        return 0

    lax.fori_loop(0, nreal, issue, 0)

    padv = jnp.full((16,), N, jnp.int32)

    def fill(j, _):
        for c in range(K // 16):
            col_v[j, pl.ds(c * 16, 16)] = padv
        return 0

    lax.fori_loop(nreal, nchunk, fill, 0)


def _fire_row(ei_hbm, row_v, wid, nchunk, nreal, sem):
    base0 = wid * nchunk * K

    def issue(j, _):
        pltpu.async_copy(
            ei_hbm.at[0, pl.ds(base0 + j * K, K)],
            row_v.at[pl.ds(j * K, K)], sem)
        return 0

    lax.fori_loop(0, nreal, issue, 0)

    padv = jnp.full((16,), N, jnp.int32)

    def fill(j, _):
        for c in range(K // 16):
            row_v[pl.ds(j * K + c * 16, 16)] = padv
        return 0

    lax.fori_loop(nreal, nchunk, fill, 0)


def _drain_idx(ei_hbm, dst_slice, nreal, sem):
    def drain(j, _):
        pltpu.make_async_copy(
            ei_hbm.at[0, pl.ds(0, K)], dst_slice, sem).wait()
        return 0

    lax.fori_loop(0, nreal, drain, 0)


def _make_deg_kernel(nchunk, e):
    @functools.partial(
        pl.kernel,
        out_type=jax.ShapeDtypeStruct((NC, N_PAD, DEGW), jnp.float32),
        mesh=_mesh,
        scratch_types=[
            pltpu.VMEM((nchunk, K), jnp.int32),
            pltpu.VMEM((K, DEGW), jnp.float32),
            pltpu.VMEM((RPT, DEGW), jnp.float32),
            pltpu.VMEM_SHARED((N_PAD, DEGW), jnp.float32),
            pltpu.SemaphoreType.DMA,
        ],
        compiler_params=pltpu.CompilerParams(use_tc_tiling_on_sc=False),
    )
    def deg_kernel(ei_hbm, degp_hbm, col_v, ones_v, zrow_v, deg_sh, dsem):
        cid = lax.axis_index("c")
        sid = lax.axis_index("s")
        wid = sid * NC + cid
        nreal = _nreal(e, wid, nchunk)
        _fire_col(ei_hbm, col_v, wid, nchunk, nreal, dsem)

        one = jnp.full((16,), 1.0, jnp.float32)

        def fill_ones(i, _):
            ones_v[i, :] = one
            return 0

        lax.fori_loop(0, K, fill_ones, 0)
        _zero_rows(zrow_v, RPT, DEGW)

        pltpu.sync_copy(zrow_v, deg_sh.at[pl.ds(sid * RPT, RPT)])
        _drain_idx(ei_hbm, col_v.at[0], nreal, dsem)
        plsc.subcore_barrier()

        for j in range(8):
            pltpu.async_copy(ones_v, deg_sh.at[col_v.at[j]], dsem, add=True)

        def chunk(h, _):
            for j in range(8):
                s = 8 * h + j
                pltpu.make_async_copy(
                    ones_v, deg_sh.at[col_v.at[s]], dsem).wait()
                pltpu.async_copy(
                    ones_v, deg_sh.at[col_v.at[s + 8]], dsem, add=True)
            return 0

        lax.fori_loop(0, nchunk // 8 - 1, chunk, 0)
        for j in range(8):
            pltpu.make_async_copy(
                ones_v, deg_sh.at[col_v.at[j]], dsem).wait()
        plsc.subcore_barrier()

        pltpu.sync_copy(
            deg_sh.at[pl.ds(sid * RPT, RPT)],
            degp_hbm.at[cid, pl.ds(sid * RPT, RPT)],
        )

    return deg_kernel


def _ring(feat_sh, agg_sh, row_v, col_v, bufs, zrow_v, gsem, ssem, nchunk):
    def ridx(s):
        return row_v.at[pl.ds(s * K, K)]

    zsrc = zrow_v.at[pl.ds(0, K)]
    for j in range(HALF):
        pltpu.async_copy(
            zsrc, agg_sh.at[col_v.at[0]], ssem[HALF + j], add=True)
    for j in range(HALF):
        pltpu.async_copy(feat_sh.at[ridx(j)], bufs[j], gsem[j])

    def step(h, _):
        for j in range(NBUF):
            s = NBUF * h + j
            pltpu.make_async_copy(
                feat_sh.at[ridx(s)], bufs[j], gsem[j]).wait()
            pltpu.async_copy(
                bufs[j], agg_sh.at[col_v.at[s]], ssem[j], add=True)
            bb = (j + HALF) % NBUF
            pltpu.make_async_copy(
                bufs[bb], agg_sh.at[col_v.at[s]], ssem[bb]).wait()
            s2 = (s + HALF) % nchunk
            pltpu.async_copy(feat_sh.at[ridx(s2)], bufs[bb], gsem[bb])
        return 0

    lax.fori_loop(0, nchunk // NBUF, step, 0)

    for j in range(HALF):
        pltpu.make_async_copy(
            bufs[HALF + j], agg_sh.at[col_v.at[0]], ssem[HALF + j]).wait()
        pltpu.make_async_copy(
            feat_sh.at[ridx(j)], bufs[j], gsem[j]).wait()


def _make_agg1_kernel(nchunk, e):
    @functools.partial(
        pl.kernel,
        out_type=jax.ShapeDtypeStruct((NC, N_PAD, HID), jnp.float32),
        mesh=_mesh,
        scratch_types=[
            pltpu.VMEM((nchunk * K,), jnp.int32),
            pltpu.VMEM((nchunk, K), jnp.int32),
            [pltpu.VMEM((K, HID), jnp.float32) for _ in range(NBUF)],
            pltpu.VMEM((RPT, HID), jnp.float32),
            pltpu.VMEM_SHARED((N_PAD, HID), jnp.float32),
            pltpu.VMEM_SHARED((N_PAD, HID), jnp.float32),
            [pltpu.SemaphoreType.DMA for _ in range(NBUF)],
            [pltpu.SemaphoreType.DMA for _ in range(NBUF)],
        ],
        compiler_params=pltpu.CompilerParams(use_tc_tiling_on_sc=False),
    )
    def agg_kernel(feat_hbm, ei_hbm, aggp_hbm,
                   row_v, col_v, bufs, zrow_v, agg_sh, feat_sh, gsem, ssem):
        cid = lax.axis_index("c")
        sid = lax.axis_index("s")
        wid = sid * NC + cid
        nreal = _nreal(e, wid, nchunk)

        _fire_row(ei_hbm, row_v, wid, nchunk, nreal, gsem[0])
        _fire_col(ei_hbm, col_v, wid, nchunk, nreal, gsem[1])

        pltpu.sync_copy(
            feat_hbm.at[pl.ds(sid * RPT, RPT)],
            feat_sh.at[pl.ds(sid * RPT, RPT)],
        )

        _zero_rows(zrow_v, RPT, HID)
        pltpu.sync_copy(zrow_v, agg_sh.at[pl.ds(sid * RPT, RPT)])
        _drain_idx(ei_hbm, row_v.at[pl.ds(0, K)], nreal, gsem[0])
        _drain_idx(ei_hbm, col_v.at[0], nreal, gsem[1])
        plsc.subcore_barrier()

        _ring(feat_sh, agg_sh, row_v, col_v, bufs, zrow_v, gsem, ssem, nchunk)
        plsc.subcore_barrier()

        pltpu.sync_copy(
            agg_sh.at[pl.ds(sid * RPT, RPT)],
            aggp_hbm.at[cid, pl.ds(sid * RPT, RPT)],
        )

    return agg_kernel


def _make_agg2_kernel(nchunk, e):
    @functools.partial(
        pl.kernel,
        out_type=jax.ShapeDtypeStruct((NC, N_PAD, HID), jnp.float32),
        mesh=_mesh,
        scratch_types=[
            pltpu.VMEM((nchunk * K,), jnp.int32),
            pltpu.VMEM((nchunk, K), jnp.int32),
            [pltpu.VMEM((K, HID), jnp.float32) for _ in range(NBUF)],
            pltpu.VMEM((RPT, HID), jnp.float32),
            pltpu.VMEM((HID,), jnp.float32),
            pltpu.VMEM_SHARED((N_PAD, HID), jnp.float32),
            pltpu.VMEM_SHARED((N_PAD, HID), jnp.float32),
            [pltpu.SemaphoreType.DMA for _ in range(NBUF)],
            [pltpu.SemaphoreType.DMA for _ in range(NBUF)],
        ],
        compiler_params=pltpu.CompilerParams(use_tc_tiling_on_sc=False),
    )
    def agg_kernel(aggp_in_hbm, dinvw_hbm, b1_hbm, ei_hbm,
                   aggp_hbm, row_v, col_v, bufs, zrow_v, b1_v,
                   agg_sh, feat_sh, gsem, ssem):
        cid = lax.axis_index("c")
        sid = lax.axis_index("s")
        wid = sid * NC + cid
        nreal = _nreal(e, wid, nchunk)

        _fire_row(ei_hbm, row_v, wid, nchunk, nreal, gsem[7])
        _fire_col(ei_hbm, col_v, wid, nchunk, nreal, gsem[3])
        pltpu.sync_copy(b1_hbm, b1_v)

        def piece_srcs(p):
            base = sid * RPT + p * K
            return (aggp_in_hbm.at[0, pl.ds(base, K)],
                    aggp_in_hbm.at[1, pl.ds(base, K)],
                    dinvw_hbm.at[pl.ds(base, K)])

        def fire_piece(p):
            g = (p % 2) * 4
            for q, src in enumerate(piece_srcs(p)):
                pltpu.async_copy(src, bufs[g + q], gsem[g + q])

        fire_piece(0)
        fire_piece(1)
        _zero_rows(zrow_v, RPT, HID)

        for p in range(PIECES):
            g = (p % 2) * 4
            base = sid * RPT + p * K
            for q, src in enumerate(piece_srcs(p)):
                pltpu.make_async_copy(src, bufs[g + q], gsem[g + q]).wait()
            if p >= 2:
                pltpu.make_async_copy(
                    bufs[g + 3], feat_sh.at[pl.ds(0, K)],
                    ssem[p % 2]).wait()

            def row(i, _):
                for c in range(HID // 16):
                    sl = pl.ds(c * 16, 16)
                    d = bufs[g + 2][i, sl]
                    a = bufs[g][i, sl] + bufs[g + 1][i, sl]
                    h = jnp.maximum(a * d + b1_v[sl], 0.0)
                    bufs[g + 3][i, sl] = h * d
                return 0

            lax.fori_loop(0, K, row, 0)
            pltpu.async_copy(
                bufs[g + 3], feat_sh.at[pl.ds(base, K)], ssem[p % 2])
            if p + 2 < PIECES:
                fire_piece(p + 2)

        for p in (PIECES - 2, PIECES - 1):
            g = (p % 2) * 4
            pltpu.make_async_copy(
                bufs[g + 3], feat_sh.at[pl.ds(0, K)], ssem[p % 2]).wait()

        pltpu.sync_copy(zrow_v, agg_sh.at[pl.ds(sid * RPT, RPT)])
        _drain_idx(ei_hbm, row_v.at[pl.ds(0, K)], nreal, gsem[7])
        _drain_idx(ei_hbm, col_v.at[0], nreal, gsem[3])
        plsc.subcore_barrier()

        _ring(feat_sh, agg_sh, row_v, col_v, bufs, zrow_v, gsem, ssem, nchunk)
        plsc.subcore_barrier()

        pltpu.sync_copy(
            agg_sh.at[pl.ds(sid * RPT, RPT)],
            aggp_hbm.at[cid, pl.ds(sid * RPT, RPT)],
        )

    return agg_kernel


def _tc_xw_scale(x, w1, degp):
    def body(x_ref, w_ref, degp_ref, t1_ref, dinv_ref):
        deg = degp_ref[0, :, 0:1] + degp_ref[1, :, 0:1]
        rows = lax.broadcasted_iota(jnp.int32, (N_PAD, 1), 0)
        dinv = jnp.where(
            (deg > 0) & (rows < N),
            lax.rsqrt(jnp.maximum(deg, 1e-12)), 0.0)
        w_wide = jnp.concatenate([w_ref[...]] * 4, axis=1)
        xw = jnp.dot(x_ref[...], w_wide,
                     preferred_element_type=jnp.float32)
        xww = jnp.concatenate(
            [xw, jnp.zeros((N_PAD - N, 128), jnp.float32)], axis=0) * dinv
        lane = lax.broadcasted_iota(jnp.int32, (1, 128), 1)
        x3 = xww.reshape(FN, 4, 128)
        dv3 = jnp.broadcast_to(dinv, (N_PAD, 128)).reshape(FN, 4, 128)
        t1 = jnp.zeros((FN, 128), jnp.float32)
        dw = jnp.zeros((FN, 128), jnp.float32)
        for c in range(4):
            lm = ((lane >= 32 * c) & (lane < 32 * (c + 1))).astype(
                jnp.float32)
            t1 = t1 + x3[:, c, :] * lm
            dw = dw + dv3[:, c, :] * lm
        t1_ref[...] = t1
        dinv_ref[...] = dw

    return pl.pallas_call(
        body,
        out_shape=(
            jax.ShapeDtypeStruct((FN, 128), jnp.float32),
            jax.ShapeDtypeStruct((FN, 128), jnp.float32),
        ),
    )(x, w1, degp)


FB = 2048
FBF = FB // 4


def _tc_final(aggp_f, dinvw_f, w2, b2):
    def body(aggp_ref, dinv_ref, w2_ref, b2_ref, out_ref):
        z = (aggp_ref[0] + aggp_ref[1]) * dinv_ref[...]
        rep = jnp.broadcast_to(
            z.reshape(FBF, 1, 128), (FBF, 4, 128)).reshape(FB, 128)
        phase = lax.broadcasted_iota(jnp.int32, (FB, 1), 0) % 4
        yin = jnp.zeros((FB, HID), jnp.float32)
        for c in range(4):
            yin = yin + jnp.where(
                phase == c, rep[:, 32 * c:32 * (c + 1)], 0.0)
        y = jnp.dot(yin, w2_ref[...],
                    preferred_element_type=jnp.float32) + b2_ref[...]
        m = jnp.max(y, axis=1, keepdims=True)
        s = y - m
        lse = jnp.log(jnp.sum(jnp.exp(s), axis=1, keepdims=True))
        out_ref[...] = s - lse

    return pl.pallas_call(
        body,
        grid=(N_PAD // FB,),
        in_specs=[
            pl.BlockSpec((NC, FBF, 128), lambda g: (0, g, 0)),
            pl.BlockSpec((FBF, 128), lambda g: (g, 0)),
            pl.BlockSpec((HID, 128), lambda g: (0, 0)),
            pl.BlockSpec((128,), lambda g: (0,)),
        ],
        out_specs=pl.BlockSpec((FB, 128), lambda g: (g, 0)),
        out_shape=jax.ShapeDtypeStruct((N, 128), jnp.float32),
    )(aggp_f, dinvw_f, w2, b2)


def kernel(x, edge_index, W1, b1, W2, b2):
    e = edge_index.shape[1]
    nchunk = -(-e // (NW * K))
    nchunk = -(-nchunk // NBUF) * NBUF

    ei = edge_index.astype(jnp.int32)
    if e % K:
        pad = jnp.full((2, K - e % K), N, jnp.int32)
        ei = jnp.concatenate([ei, pad], axis=1)
        e = ei.shape[1]

    degp = _make_deg_kernel(nchunk, e)(ei)
    t1s_f, dinvw_f = _tc_xw_scale(x, W1, degp)
    agg1p = _make_agg1_kernel(nchunk, e)(t1s_f.reshape(N_PAD, HID), ei)
    agg2p = _make_agg2_kernel(nchunk, e)(
        agg1p, dinvw_f.reshape(N_PAD, HID), b1, ei)
    return _tc_final(agg2p.reshape(NC, FN, 128), dinvw_f, W2, b2)


# speedup vs baseline: 1.2709x; 1.0133x over previous
import functools

import jax
import jax.numpy as jnp
from jax import lax
from jax.experimental import pallas as pl
from jax.experimental.pallas import tpu as pltpu
from jax.experimental.pallas import tpu_sc as plsc

N = 10000
HID = 32
DEGW = 16

NC = 2
NS = 16
NW = NC * NS
K = 128
NBUF = 10
HALF = NBUF // 2

N_PAD = 10240
RPT = N_PAD // NS
FN = N_PAD * HID // 128
PIECES = RPT // K

_mesh = plsc.VectorSubcoreMesh(core_axis_name="c", subcore_axis_name="s")


def _zero_rows(ref, nrows, ncols):
    zero = jnp.zeros((16,), jnp.float32)

    def body(i, _):
        for c in range(ncols // 16):
            ref[i, pl.ds(c * 16, 16)] = zero
        return 0

    lax.fori_loop(0, nrows, body, 0)


def _nreal(e, wid, nchunk):
    return jnp.clip(e // K - wid * nchunk, 0, nchunk)


def _fire_col(ei_hbm, col_v, wid, nchunk, nreal, sem):
    base0 = wid * nchunk * K

    def issue(j, _):
        pltpu.async_copy(
            ei_hbm.at[1, pl.ds(base0 + j * K, K)], col_v.at[j], sem)
        return 0

    lax.fori_loop(0, nreal, issue, 0)

    padv = jnp.full((16,), N, jnp.int32)

    def fill(j, _):
        for c in range(K // 16):
            col_v[j, pl.ds(c * 16, 16)] = padv
        return 0

    lax.fori_loop(nreal, nchunk, fill, 0)


def _fire_row(ei_hbm, row_v, wid, nchunk, nreal, sem):
    base0 = wid * nchunk * K

    def issue(j, _):
        pltpu.async_copy(
            ei_hbm.at[0, pl.ds(base0 + j * K, K)],
            row_v.at[pl.ds(j * K, K)], sem)
        return 0

    lax.fori_loop(0, nreal, issue, 0)

    padv = jnp.full((16,), N, jnp.int32)

    def fill(j, _):
        for c in range(K // 16):
            row_v[pl.ds(j * K + c * 16, 16)] = padv
        return 0

    lax.fori_loop(nreal, nchunk, fill, 0)


def _drain_idx(ei_hbm, dst_slice, nreal, sem):
    def drain(j, _):
        pltpu.make_async_copy(
            ei_hbm.at[0, pl.ds(0, K)], dst_slice, sem).wait()
        return 0

    lax.fori_loop(0, nreal, drain, 0)


def _make_deg_kernel(nchunk, e):
    @functools.partial(
        pl.kernel,
        out_type=jax.ShapeDtypeStruct((NC, N_PAD, DEGW), jnp.float32),
        mesh=_mesh,
        scratch_types=[
            pltpu.VMEM((nchunk, K), jnp.int32),
            pltpu.VMEM((K, DEGW), jnp.float32),
            pltpu.VMEM((RPT, DEGW), jnp.float32),
            pltpu.VMEM_SHARED((N_PAD, DEGW), jnp.float32),
            pltpu.SemaphoreType.DMA,
        ],
        compiler_params=pltpu.CompilerParams(use_tc_tiling_on_sc=False),
    )
    def deg_kernel(ei_hbm, degp_hbm, col_v, ones_v, zrow_v, deg_sh, dsem):
        cid = lax.axis_index("c")
        sid = lax.axis_index("s")
        wid = sid * NC + cid
        nreal = _nreal(e, wid, nchunk)
        _fire_col(ei_hbm, col_v, wid, nchunk, nreal, dsem)

        one = jnp.full((16,), 1.0, jnp.float32)

        def fill_ones(i, _):
            ones_v[i, :] = one
            return 0

        lax.fori_loop(0, K, fill_ones, 0)
        _zero_rows(zrow_v, RPT, DEGW)

        pltpu.sync_copy(zrow_v, deg_sh.at[pl.ds(sid * RPT, RPT)])
        _drain_idx(ei_hbm, col_v.at[0], nreal, dsem)
        plsc.subcore_barrier()

        for j in range(8):
            pltpu.async_copy(ones_v, deg_sh.at[col_v.at[j]], dsem, add=True)

        def chunk(h, _):
            for j in range(8):
                s = 8 * h + j
                pltpu.make_async_copy(
                    ones_v, deg_sh.at[col_v.at[s]], dsem).wait()
                pltpu.async_copy(
                    ones_v, deg_sh.at[col_v.at[s + 8]], dsem, add=True)
            return 0

        lax.fori_loop(0, nchunk // 8 - 1, chunk, 0)
        for j in range(8):
            pltpu.make_async_copy(
                ones_v, deg_sh.at[col_v.at[j]], dsem).wait()
        plsc.subcore_barrier()

        pltpu.sync_copy(
            deg_sh.at[pl.ds(sid * RPT, RPT)],
            degp_hbm.at[cid, pl.ds(sid * RPT, RPT)],
        )

    return deg_kernel


def _ring(feat_sh, agg_sh, row_v, col_v, bufs, zrow_v, gsem, ssem, nchunk):
    def ridx(s):
        return row_v.at[pl.ds(s * K, K)]

    zsrc = zrow_v.at[pl.ds(0, K)]
    for j in range(HALF):
        pltpu.async_copy(
            zsrc, agg_sh.at[col_v.at[0]], ssem[HALF + j], add=True)
    for j in range(HALF):
        pltpu.async_copy(feat_sh.at[ridx(j)], bufs[j], gsem[j])

    def step(h, _):
        for j in range(NBUF):
            s = NBUF * h + j
            pltpu.make_async_copy(
                feat_sh.at[ridx(s)], bufs[j], gsem[j]).wait()
            pltpu.async_copy(
                bufs[j], agg_sh.at[col_v.at[s]], ssem[j], add=True)
            bb = (j + HALF) % NBUF
            pltpu.make_async_copy(
                bufs[bb], agg_sh.at[col_v.at[s]], ssem[bb]).wait()
            s2 = (s + HALF) % nchunk
            pltpu.async_copy(feat_sh.at[ridx(s2)], bufs[bb], gsem[bb])
        return 0

    lax.fori_loop(0, nchunk // NBUF, step, 0)

    for j in range(HALF):
        pltpu.make_async_copy(
            bufs[HALF + j], agg_sh.at[col_v.at[0]], ssem[HALF + j]).wait()
        pltpu.make_async_copy(
            feat_sh.at[ridx(j)], bufs[j], gsem[j]).wait()


def _make_agg1_kernel(nchunk, e):
    @functools.partial(
        pl.kernel,
        out_type=jax.ShapeDtypeStruct((NC, N_PAD, HID), jnp.float32),
        mesh=_mesh,
        scratch_types=[
            pltpu.VMEM((nchunk * K,), jnp.int32),
            pltpu.VMEM((nchunk, K), jnp.int32),
            [pltpu.VMEM((K, HID), jnp.float32) for _ in range(NBUF)],
            pltpu.VMEM((RPT, HID), jnp.float32),
            pltpu.VMEM_SHARED((N_PAD, HID), jnp.float32),
            pltpu.VMEM_SHARED((N_PAD, HID), jnp.float32),
            [pltpu.SemaphoreType.DMA for _ in range(NBUF)],
            [pltpu.SemaphoreType.DMA for _ in range(NBUF)],
        ],
        compiler_params=pltpu.CompilerParams(use_tc_tiling_on_sc=False),
    )
    def agg_kernel(feat_hbm, ei_hbm, aggp_hbm,
                   row_v, col_v, bufs, zrow_v, agg_sh, feat_sh, gsem, ssem):
        cid = lax.axis_index("c")
        sid = lax.axis_index("s")
        wid = sid * NC + cid
        nreal = _nreal(e, wid, nchunk)

        _fire_row(ei_hbm, row_v, wid, nchunk, nreal, gsem[0])
        _fire_col(ei_hbm, col_v, wid, nchunk, nreal, gsem[1])

        pltpu.sync_copy(
            feat_hbm.at[pl.ds(sid * RPT, RPT)],
            feat_sh.at[pl.ds(sid * RPT, RPT)],
        )

        _zero_rows(zrow_v, RPT, HID)
        pltpu.sync_copy(zrow_v, agg_sh.at[pl.ds(sid * RPT, RPT)])
        _drain_idx(ei_hbm, row_v.at[pl.ds(0, K)], nreal, gsem[0])
        _drain_idx(ei_hbm, col_v.at[0], nreal, gsem[1])
        plsc.subcore_barrier()

        _ring(feat_sh, agg_sh, row_v, col_v, bufs, zrow_v, gsem, ssem, nchunk)
        plsc.subcore_barrier()

        pltpu.sync_copy(
            agg_sh.at[pl.ds(sid * RPT, RPT)],
            aggp_hbm.at[cid, pl.ds(sid * RPT, RPT)],
        )

    return agg_kernel


def _make_agg2_kernel(nchunk, e):
    @functools.partial(
        pl.kernel,
        out_type=jax.ShapeDtypeStruct((NC, N_PAD, HID), jnp.float32),
        mesh=_mesh,
        scratch_types=[
            pltpu.VMEM((nchunk * K,), jnp.int32),
            pltpu.VMEM((nchunk, K), jnp.int32),
            [pltpu.VMEM((K, HID), jnp.float32) for _ in range(NBUF)],
            pltpu.VMEM((RPT, HID), jnp.float32),
            pltpu.VMEM((HID,), jnp.float32),
            pltpu.VMEM_SHARED((N_PAD, HID), jnp.float32),
            pltpu.VMEM_SHARED((N_PAD, HID), jnp.float32),
            [pltpu.SemaphoreType.DMA for _ in range(NBUF)],
            [pltpu.SemaphoreType.DMA for _ in range(NBUF)],
        ],
        compiler_params=pltpu.CompilerParams(use_tc_tiling_on_sc=False),
    )
    def agg_kernel(aggp_in_hbm, dinvw_hbm, b1_hbm, ei_hbm,
                   aggp_hbm, row_v, col_v, bufs, zrow_v, b1_v,
                   agg_sh, feat_sh, gsem, ssem):
        cid = lax.axis_index("c")
        sid = lax.axis_index("s")
        wid = sid * NC + cid
        nreal = _nreal(e, wid, nchunk)

        _fire_row(ei_hbm, row_v, wid, nchunk, nreal, gsem[7])
        _fire_col(ei_hbm, col_v, wid, nchunk, nreal, gsem[3])
        pltpu.sync_copy(b1_hbm, b1_v)

        def piece_srcs(p):
            base = sid * RPT + p * K
            return (aggp_in_hbm.at[0, pl.ds(base, K)],
                    aggp_in_hbm.at[1, pl.ds(base, K)],
                    dinvw_hbm.at[pl.ds(base, K)])

        def fire_piece(p):
            g = (p % 2) * 4
            for q, src in enumerate(piece_srcs(p)):
                pltpu.async_copy(src, bufs[g + q], gsem[g + q])

        fire_piece(0)
        fire_piece(1)
        _zero_rows(zrow_v, RPT, HID)

        for p in range(PIECES):
            g = (p % 2) * 4
            base = sid * RPT + p * K
            for q, src in enumerate(piece_srcs(p)):
                pltpu.make_async_copy(src, bufs[g + q], gsem[g + q]).wait()
            if p >= 2:
                pltpu.make_async_copy(
                    bufs[g + 3], feat_sh.at[pl.ds(0, K)],
                    ssem[p % 2]).wait()

            def row(i, _):
                for c in range(HID // 16):
                    sl = pl.ds(c * 16, 16)
                    d = bufs[g + 2][i, sl]
                    a = bufs[g][i, sl] + bufs[g + 1][i, sl]
                    h = jnp.maximum(a * d + b1_v[sl], 0.0)
                    bufs[g + 3][i, sl] = h * d
                return 0

            lax.fori_loop(0, K, row, 0)
            pltpu.async_copy(
                bufs[g + 3], feat_sh.at[pl.ds(base, K)], ssem[p % 2])
            if p + 2 < PIECES:
                fire_piece(p + 2)

        for p in (PIECES - 2, PIECES - 1):
            g = (p % 2) * 4
            pltpu.make_async_copy(
                bufs[g + 3], feat_sh.at[pl.ds(0, K)], ssem[p % 2]).wait()

        pltpu.sync_copy(zrow_v, agg_sh.at[pl.ds(sid * RPT, RPT)])
        _drain_idx(ei_hbm, row_v.at[pl.ds(0, K)], nreal, gsem[7])
        _drain_idx(ei_hbm, col_v.at[0], nreal, gsem[3])
        plsc.subcore_barrier()

        _ring(feat_sh, agg_sh, row_v, col_v, bufs, zrow_v, gsem, ssem, nchunk)
        plsc.subcore_barrier()

        pltpu.sync_copy(
            agg_sh.at[pl.ds(sid * RPT, RPT)],
            aggp_hbm.at[cid, pl.ds(sid * RPT, RPT)],
        )

    return agg_kernel


def _tc_xw_scale(x, w1, degp):
    def body(x_ref, w_ref, degp_ref, t1_ref, dinv_ref):
        g = pl.program_id(0)
        deg = degp_ref[0, :, 0:1] + degp_ref[1, :, 0:1]
        rows = g * FB + lax.broadcasted_iota(jnp.int32, (FB, 1), 0)
        keep = rows < N
        dinv = jnp.where(
            (deg > 0) & keep,
            lax.rsqrt(jnp.maximum(deg, 1e-12)), 0.0)
        w_wide = jnp.concatenate([w_ref[...]] * 4, axis=1)
        xw = jnp.dot(x_ref[...], w_wide,
                     preferred_element_type=jnp.float32)
        xww = jnp.where(keep, xw, 0.0) * dinv
        lane = lax.broadcasted_iota(jnp.int32, (1, 128), 1)
        x3 = xww.reshape(FBF, 4, 128)
        dv3 = jnp.broadcast_to(dinv, (FB, 128)).reshape(FBF, 4, 128)
        t1 = jnp.zeros((FBF, 128), jnp.float32)
        dw = jnp.zeros((FBF, 128), jnp.float32)
        for c in range(4):
            lm = ((lane >= 32 * c) & (lane < 32 * (c + 1))).astype(
                jnp.float32)
            t1 = t1 + x3[:, c, :] * lm
            dw = dw + dv3[:, c, :] * lm
        t1_ref[...] = t1
        dinv_ref[...] = dw

    fshape = jax.ShapeDtypeStruct((FN, 128), jnp.float32)
    return pl.pallas_call(
        body,
        grid=(N_PAD // FB,),
        in_specs=[
            pl.BlockSpec((FB, 128), lambda g: (g, 0)),
            pl.BlockSpec((128, HID), lambda g: (0, 0)),
            pl.BlockSpec((NC, FB, DEGW), lambda g: (0, g, 0)),
        ],
        out_specs=[
            pl.BlockSpec((FBF, 128), lambda g: (g, 0)),
            pl.BlockSpec((FBF, 128), lambda g: (g, 0)),
        ],
        out_shape=(fshape, fshape),
    )(x, w1, degp)


FB = 2048
FBF = FB // 4


def _tc_final(aggp_f, dinvw_f, w2, b2):
    def body(aggp_ref, dinv_ref, w2_ref, b2_ref, out_ref):
        z = (aggp_ref[0] + aggp_ref[1]) * dinv_ref[...]
        rep = jnp.broadcast_to(
            z.reshape(FBF, 1, 128), (FBF, 4, 128)).reshape(FB, 128)
        phase = lax.broadcasted_iota(jnp.int32, (FB, 1), 0) % 4
        yin = jnp.zeros((FB, HID), jnp.float32)
        for c in range(4):
            yin = yin + jnp.where(
                phase == c, rep[:, 32 * c:32 * (c + 1)], 0.0)
        y = jnp.dot(yin, w2_ref[...],
                    preferred_element_type=jnp.float32) + b2_ref[...]
        m = jnp.max(y, axis=1, keepdims=True)
        s = y - m
        lse = jnp.log(jnp.sum(jnp.exp(s), axis=1, keepdims=True))
        out_ref[...] = s - lse

    return pl.pallas_call(
        body,
        grid=(N_PAD // FB,),
        in_specs=[
            pl.BlockSpec((NC, FBF, 128), lambda g: (0, g, 0)),
            pl.BlockSpec((FBF, 128), lambda g: (g, 0)),
            pl.BlockSpec((HID, 128), lambda g: (0, 0)),
            pl.BlockSpec((128,), lambda g: (0,)),
        ],
        out_specs=pl.BlockSpec((FB, 128), lambda g: (g, 0)),
        out_shape=jax.ShapeDtypeStruct((N, 128), jnp.float32),
    )(aggp_f, dinvw_f, w2, b2)


def kernel(x, edge_index, W1, b1, W2, b2):
    e = edge_index.shape[1]
    nchunk = -(-e // (NW * K))
    nchunk = -(-nchunk // NBUF) * NBUF

    ei = edge_index.astype(jnp.int32)
    if e % K:
        pad = jnp.full((2, K - e % K), N, jnp.int32)
        ei = jnp.concatenate([ei, pad], axis=1)
        e = ei.shape[1]

    degp = _make_deg_kernel(nchunk, e)(ei)
    t1s_f, dinvw_f = _tc_xw_scale(x, W1, degp)
    agg1p = _make_agg1_kernel(nchunk, e)(t1s_f.reshape(N_PAD, HID), ei)
    agg2p = _make_agg2_kernel(nchunk, e)(
        agg1p, dinvw_f.reshape(N_PAD, HID), b1, ei)
    return _tc_final(agg2p.reshape(NC, FN, 128), dinvw_f, W2, b2)
